# Initial kernel scaffold; baseline (speedup 1.0000x reference)
import jax
import jax.numpy as jnp
from jax.experimental import pallas as pl


def kernel(user_representations, targets, mu_table, sigma_table, bias_table):
    raise NotImplementedError("write your pallas kernel here")


# speedup vs baseline: 5.7477x; 5.7477x over previous
import jax
import jax.numpy as jnp
from jax import lax
from jax.experimental import pallas as pl
from jax.experimental.pallas import tpu as pltpu
from jax.experimental.pallas import tpu_sc as plsc

BATCH = 4096
SEQ_LEN = 50
EMBED_DIM = 64

NUM_WORKERS = 32
EPW = BATCH // NUM_WORKERS
CHUNK = 8
NCHUNKS = EPW // CHUNK
CP = CHUNK * SEQ_LEN
GATHER_N = 80
NGATHER = CP // GATHER_N
NGROUP = CP // 16


def _body(user_hbm, tgt_hbm, mu_hbm, bias_hbm, out_hbm,
          t_v, rows_v, u_v, bias_v, out_v, sem):
    wid = lax.axis_index("s") * 2 + lax.axis_index("c")

    @pl.loop(0, NCHUNKS)
    def _chunk(c):
        base_e = wid * EPW + c * CHUNK
        tbase = base_e * SEQ_LEN

        for g in range(NGATHER):
            pltpu.sync_copy(
                tgt_hbm.at[pl.ds(tbase + g * GATHER_N, GATHER_N)], t_v.at[g])

        copies = []
        for g in range(NGATHER):
            copies.append(pltpu.async_copy(
                mu_hbm.at[t_v.at[g]],
                rows_v.at[pl.ds(g * GATHER_N, GATHER_N)], sem))
            copies.append(pltpu.async_copy(
                bias_hbm.at[t_v.at[g]],
                bias_v.at[pl.ds(g * GATHER_N, GATHER_N)], sem))
        u_copy = pltpu.async_copy(user_hbm.at[pl.ds(base_e, CHUNK)], u_v, sem)
        for cp in copies:
            cp.wait()
        u_copy.wait()

        @pl.loop(0, NGROUP)
        def _group(g16):
            p_v = lax.iota(jnp.int32, 16) + g16 * 16
            e_v = lax.div(p_v, SEQ_LEN)
            l_v = p_v - e_v * SEQ_LEN
            acc = jnp.zeros((16,), jnp.float32)
            for d in range(EMBED_DIM):
                d_v = jnp.full((16,), d, jnp.int32)
                m = plsc.load_gather(rows_v, [p_v, d_v])
                u = plsc.load_gather(u_v, [e_v, d_v, l_v])
                acc = acc + m * u
            b = plsc.load_gather(bias_v, [p_v])
            out_v[pl.ds(g16 * 16, 16)] = acc + b

        pltpu.sync_copy(out_v, out_hbm.at[pl.ds(tbase, CP)])


@jax.jit
def kernel(user_representations, targets, mu_table, sigma_table, bias_table):
    del sigma_table
    tgt_flat = targets.reshape(BATCH * SEQ_LEN).astype(jnp.int32)
    bias_flat = bias_table.reshape(-1)

    mesh = plsc.VectorSubcoreMesh(core_axis_name="c", subcore_axis_name="s")
    run = pl.kernel(
        _body,
        out_type=jax.ShapeDtypeStruct((BATCH * SEQ_LEN,), jnp.float32),
        mesh=mesh,
        compiler_params=pltpu.CompilerParams(
            use_tc_tiling_on_sc=False, needs_layout_passes=False),
        scratch_types=[
            pltpu.VMEM((NGATHER, GATHER_N), jnp.int32),
            pltpu.VMEM((CP, EMBED_DIM), jnp.float32),
            pltpu.VMEM((CHUNK, EMBED_DIM, SEQ_LEN), jnp.float32),
            pltpu.VMEM((CP,), jnp.float32),
            pltpu.VMEM((CP,), jnp.float32),
            pltpu.SemaphoreType.DMA,
        ],
    )
    out_flat = run(user_representations, tgt_flat, mu_table, bias_flat)
    return out_flat.reshape(BATCH, SEQ_LEN)
